# baseline (speedup 1.0000x reference)
import functools

import jax
import jax.numpy as jnp
from jax import lax
from jax.experimental import pallas as pl
from jax.experimental.pallas import tpu as pltpu
from jax.experimental.pallas import tpu_sc as plsc

B = 4096
L = 20
V = 100000
D = 64
V2 = V // 2

NC = 2
NS = 16
NW = NC * NS
BAGS_W = B // NW
CHUNK = 32
NCHUNK = BAGS_W // CHUNK
IDX_CHUNK = CHUNK * L
IDX_ROWS = IDX_CHUNK // 128


def _sc_body(lt_tab, gt_tab, show_tab, lt_idx, gt_idx, show_ids, val6,
             idx_v, idxp_v, rows_v, out_v, sem):
    wid = lax.axis_index("s") * NC + lax.axis_index("c")
    bag_base = wid * BAGS_W

    def reduce_chunk(c, col0):
        def bag_body(i, _):
            rbase = i * L
            o0 = (idx_v[pl.ds(rbase, 16)] & 1) << 6
            o1 = (idx_v[pl.ds(rbase + 4, 16)] & 1) << 6
            offs = [o0[r] for r in range(16)] + [o1[12 + r] for r in range(4)]
            for d in range(4):
                m = rows_v[rbase, pl.ds(offs[0] + d * 16, 16)]
                for r in range(1, L):
                    m = jnp.maximum(
                        m, rows_v[rbase + r, pl.ds(offs[r] + d * 16, 16)])
                out_v[c * CHUNK + i, pl.ds(col0 + d * 16, 16)] = m
            return 0
        lax.fori_loop(0, CHUNK, bag_body, 0)

    def do_feature(tab, idx_flat, t, col0):
        def chunk_body(c, _):
            off = pl.multiple_of(t * (B * L) + (bag_base + c * CHUNK) * L, 8)
            pltpu.sync_copy(idx_flat.at[pl.ds(off, IDX_CHUNK)], idx_v)

            def pack_body(j, _):
                sl = pl.ds(j * 16, 16)
                idxp_v[sl] = (idx_v[sl] >> 1) + t * V2
                return 0
            lax.fori_loop(0, IDX_CHUNK // 16, pack_body, 0, unroll=4)

            cps = [
                pltpu.async_copy(tab.at[idxp_v.at[pl.ds(j * 128, 128)]],
                                 rows_v.at[pl.ds(j * 128, 128)], sem)
                for j in range(IDX_ROWS)
            ]
            for cp in cps:
                cp.wait()
            reduce_chunk(c, col0)
            return 0
        lax.fori_loop(0, NCHUNK, chunk_body, 0)

    def flush_pair(p):
        pltpu.sync_copy(
            out_v, val6.at[p, pl.ds(pl.multiple_of(bag_base, 8), BAGS_W)])

    def lt_pair(p, _):
        do_feature(lt_tab, lt_idx, 2 * p, 0)
        do_feature(lt_tab, lt_idx, 2 * p + 1, 64)
        flush_pair(p)
        return 0
    lax.fori_loop(0, 3, lt_pair, 0)

    def gt_pair(p, _):
        do_feature(gt_tab, gt_idx, 2 * p, 0)
        do_feature(gt_tab, gt_idx, 2 * p + 1, 64)
        flush_pair(3 + p)
        return 0
    lax.fori_loop(0, 2, gt_pair, 0)

    do_feature(gt_tab, gt_idx, jnp.int32(4), 0)
    pltpu.sync_copy(
        show_ids.at[pl.ds(pl.multiple_of(bag_base, 8), BAGS_W)],
        idx_v.at[pl.ds(0, BAGS_W)])
    def show_pack(j, _):
        sl = pl.ds(j * 16, 16)
        idxp_v[sl] = idx_v[sl] >> 1
        return 0
    lax.fori_loop(0, BAGS_W // 16, show_pack, 0, unroll=4)
    pltpu.async_copy(show_tab.at[idxp_v.at[pl.ds(0, BAGS_W)]],
                     rows_v.at[pl.ds(0, BAGS_W)], sem).wait()

    def show_body(g, _):
        ho = (idx_v[pl.ds(g * 16, 16)] & 1) << 6
        for r in range(16):
            i = g * 16 + r
            for d in range(4):
                out_v[i, pl.ds(64 + d * 16, 16)] = (
                    rows_v[i, pl.ds(ho[r] + d * 16, 16)])
        return 0
    lax.fori_loop(0, BAGS_W // 16, show_body, 0)
    flush_pair(5)


@jax.jit
def _sc_gather(lt_tab, gt_tab, show_tab, lt_idx, gt_idx, show_ids):
    mesh = plsc.VectorSubcoreMesh(core_axis_name="c", subcore_axis_name="s",
                                  num_cores=NC, num_subcores=NS)
    return pl.kernel(
        _sc_body,
        out_type=jax.ShapeDtypeStruct((6, B, 128), jnp.float32),
        mesh=mesh,
        scratch_types=[
            pltpu.VMEM((IDX_CHUNK,), jnp.int32),
            pltpu.VMEM((IDX_CHUNK,), jnp.int32),
            pltpu.VMEM((IDX_CHUNK, 128), jnp.float32),
            pltpu.VMEM((BAGS_W, 128), jnp.float32),
            pltpu.SemaphoreType.DMA,
        ],
    )(lt_tab, gt_tab, show_tab, lt_idx, gt_idx, show_ids)


def _mm_body(v_ref, w_ref, bias_ref, o_ref):
    acc = jnp.dot(v_ref[0], w_ref[0], preferred_element_type=jnp.float32)
    for p in range(1, 6):
        acc += jnp.dot(v_ref[p], w_ref[p], preferred_element_type=jnp.float32)
    o_ref[...] = acc + bias_ref[...]


@jax.jit
def _tc_matmul(val6, wc, bias):
    bm = 512
    return pl.pallas_call(
        _mm_body,
        grid=(B // bm,),
        in_specs=[
            pl.BlockSpec((6, bm, 128), lambda i: (0, i, 0)),
            pl.BlockSpec((6, 128, 128), lambda i: (0, 0, 0)),
            pl.BlockSpec((1, 128), lambda i: (0, 0)),
        ],
        out_specs=pl.BlockSpec((bm, 128), lambda i: (i, 0)),
        out_shape=jax.ShapeDtypeStruct((B, 128), jnp.float32),
    )(val6, wc, bias)


def kernel(lt_inputs, gt_inputs, show_ids, lt_tables, gt_tables, show_table,
           W, b):
    lt_tab = lt_tables.reshape(6 * V2, 128)
    gt_tab = gt_tables.reshape(5 * V2, 128)
    show_tab = show_table.reshape(V2, 128)
    lt_idx = lt_inputs.reshape(6 * B * L)
    gt_idx = gt_inputs.reshape(5 * B * L)
    val6 = _sc_gather(lt_tab, gt_tab, show_tab, lt_idx, gt_idx, show_ids)

    wc = W.transpose(1, 0, 2).reshape(12 * D, 25)
    wc = jnp.pad(wc, ((0, 0), (0, 103))).reshape(6, 128, 128)
    bias = jnp.pad(b.reshape(1, 25), ((0, 0), (0, 103)))
    out = _tc_matmul(val6, wc, bias)
    return out[:, :25].reshape(B, 5, 5).transpose(1, 0, 2)


# speedup vs baseline: 1.1386x; 1.1386x over previous
import functools

import jax
import jax.numpy as jnp
from jax import lax
from jax.experimental import pallas as pl
from jax.experimental.pallas import tpu as pltpu
from jax.experimental.pallas import tpu_sc as plsc

B = 4096
L = 20
V = 100000
D = 64

NC = 2
NS = 16
NW = NC * NS
BAGS_W = B // NW
CHUNK = 64
NCHUNK = BAGS_W // CHUNK
IDX_CHUNK = CHUNK * L
IDX_ROWS = IDX_CHUNK // 128


def _sc_body(lt_tab3, gt_tab3, show_tab, lt_idx2, gt_idx2, show_ids, val6,
             idx_v, rows_v, out_v, sem):
    wid = lax.axis_index("s") * NC + lax.axis_index("c")
    bag_base = wid * BAGS_W

    def reduce_chunk(c, col0):
        def bag_body(i, _):
            rbase = i * L
            for d in range(4):
                m = rows_v[rbase, pl.ds(d * 16, 16)]
                for r in range(1, L):
                    m = jnp.maximum(m, rows_v[rbase + r, pl.ds(d * 16, 16)])
                out_v[c * CHUNK + i, pl.ds(col0 + d * 16, 16)] = m
            return 0
        lax.fori_loop(0, CHUNK, bag_body, 0)

    def do_feature(tab3, idx2, t, col0):
        def chunk_body(c, _):
            off = pl.multiple_of((bag_base + c * CHUNK) * L, 8)
            pltpu.sync_copy(idx2.at[t, pl.ds(off, IDX_CHUNK)], idx_v)

            cps = [
                pltpu.async_copy(
                    tab3.at[t].at[idx_v.at[pl.ds(j * 128, 128)]],
                    rows_v.at[pl.ds(j * 128, 128)], sem)
                for j in range(IDX_ROWS)
            ]
            for cp in cps:
                cp.wait()
            reduce_chunk(c, col0)
            return 0
        lax.fori_loop(0, NCHUNK, chunk_body, 0)

    def flush_pair(p):
        pltpu.sync_copy(
            out_v, val6.at[p, pl.ds(pl.multiple_of(bag_base, 8), BAGS_W)])

    def lt_pair(p, _):
        do_feature(lt_tab3, lt_idx2, 2 * p, 0)
        do_feature(lt_tab3, lt_idx2, 2 * p + 1, 64)
        flush_pair(p)
        return 0
    lax.fori_loop(0, 3, lt_pair, 0)

    def gt_pair(p, _):
        do_feature(gt_tab3, gt_idx2, 2 * p, 0)
        do_feature(gt_tab3, gt_idx2, 2 * p + 1, 64)
        flush_pair(3 + p)
        return 0
    lax.fori_loop(0, 2, gt_pair, 0)

    do_feature(gt_tab3, gt_idx2, jnp.int32(4), 0)
    pltpu.sync_copy(
        show_ids.at[pl.ds(pl.multiple_of(bag_base, 8), BAGS_W)],
        idx_v.at[pl.ds(0, BAGS_W)])
    pltpu.async_copy(show_tab.at[idx_v.at[pl.ds(0, BAGS_W)]],
                     rows_v.at[pl.ds(0, BAGS_W)], sem).wait()

    def show_body(i, _):
        for d in range(4):
            out_v[i, pl.ds(64 + d * 16, 16)] = rows_v[i, pl.ds(d * 16, 16)]
        return 0
    lax.fori_loop(0, BAGS_W, show_body, 0, unroll=2)
    flush_pair(5)


@jax.jit
def _sc_gather(lt_tab, gt_tab, show_tab, lt_idx, gt_idx, show_ids):
    mesh = plsc.VectorSubcoreMesh(core_axis_name="c", subcore_axis_name="s",
                                  num_cores=NC, num_subcores=NS)
    return pl.kernel(
        _sc_body,
        out_type=jax.ShapeDtypeStruct((6, B, 128), jnp.float32),
        mesh=mesh,
        scratch_types=[
            pltpu.VMEM((IDX_CHUNK,), jnp.int32),
            pltpu.VMEM((IDX_CHUNK, D), jnp.float32),
            pltpu.VMEM((BAGS_W, 128), jnp.float32),
            pltpu.SemaphoreType.DMA,
        ],
        compiler_params=pltpu.CompilerParams(use_tc_tiling_on_sc=False),
    )(lt_tab, gt_tab, show_tab, lt_idx, gt_idx, show_ids)


def _mm_body(v_ref, w_ref, bias_ref, o_ref):
    acc = jnp.dot(v_ref[0], w_ref[0], preferred_element_type=jnp.float32)
    for p in range(1, 6):
        acc += jnp.dot(v_ref[p], w_ref[p], preferred_element_type=jnp.float32)
    o_ref[...] = acc + bias_ref[...]


@jax.jit
def _tc_matmul(val6, wc, bias):
    bm = 512
    return pl.pallas_call(
        _mm_body,
        grid=(B // bm,),
        in_specs=[
            pl.BlockSpec((6, bm, 128), lambda i: (0, i, 0)),
            pl.BlockSpec((6, 128, 128), lambda i: (0, 0, 0)),
            pl.BlockSpec((1, 128), lambda i: (0, 0)),
        ],
        out_specs=pl.BlockSpec((bm, 128), lambda i: (i, 0)),
        out_shape=jax.ShapeDtypeStruct((B, 128), jnp.float32),
    )(val6, wc, bias)


def kernel(lt_inputs, gt_inputs, show_ids, lt_tables, gt_tables, show_table,
           W, b):
    val6 = _sc_gather(lt_tables, gt_tables, show_table, lt_inputs, gt_inputs,
                      show_ids)
    wc = W.transpose(1, 0, 2).reshape(12 * D, 25)
    wc = jnp.pad(wc, ((0, 0), (0, 103))).reshape(6, 128, 128)
    bias = jnp.pad(b.reshape(1, 25), ((0, 0), (0, 103)))
    out = _tc_matmul(val6, wc, bias)
    return out[:, :25].reshape(B, 5, 5).transpose(1, 0, 2)
